# baseline (speedup 1.0000x reference)
import jax
import jax.numpy as jnp
from jax.experimental import pallas as pl
from jax.experimental.pallas import tpu as pltpu

_R = 6
_K = 512
_C = 192
_BN = 2048


def _rvq_stage(z_ref, cb_ref, codes_ref, zq_ref, loss_ref, usage_ref,
               res_ref, acc_ref):
    i = pl.program_id(0)
    r = pl.program_id(1)
    nb = pl.num_programs(0)

    @pl.when((i == 0) & (r == 0))
    def _init_global():
        loss_ref[...] = jnp.zeros_like(loss_ref)
        usage_ref[...] = jnp.zeros_like(usage_ref)

    @pl.when(r == 0)
    def _init_block():
        res_ref[...] = z_ref[...]
        acc_ref[...] = jnp.zeros_like(acc_ref)

    residual = res_ref[...]
    cb = cb_ref[0]
    cbn = jnp.sum(cb * cb, axis=1, keepdims=True)
    sT = jax.lax.dot_general(
        cb, residual, (((1,), (1,)), ((), ())),
        preferred_element_type=jnp.float32,
        precision=jax.lax.Precision.DEFAULT)
    dT = cbn - 2.0 * sT
    rows = jax.lax.broadcasted_iota(jnp.int32, (_K, _BN), 0)
    dmin = jnp.min(dT, axis=0, keepdims=True)
    idx = jnp.min(jnp.where(dT == dmin, rows, _K), axis=0, keepdims=True)
    onehotT = (rows == idx).astype(jnp.bfloat16)
    cb_hi = cb.astype(jnp.bfloat16)
    rem = cb - cb_hi.astype(jnp.float32)
    cb_mid = rem.astype(jnp.bfloat16)
    cb_lo = (rem - cb_mid.astype(jnp.float32)).astype(jnp.bfloat16)

    def _pick(plane):
        return jax.lax.dot_general(
            onehotT, plane, (((0,), (0,)), ((), ())),
            preferred_element_type=jnp.float32)

    q = _pick(cb_hi) + _pick(cb_mid) + _pick(cb_lo)
    res_ref[...] = residual - q
    acc_ref[...] += q
    codes_ref[0, 0, :] = idx[0]
    ones_col = jnp.ones((_BN, 1), jnp.bfloat16)
    cnt = jax.lax.dot_general(
        onehotT, ones_col, (((1,), (0,)), ((), ())),
        preferred_element_type=jnp.float32)
    usage_ref[pl.ds(r, 1)] += cnt[:, 0][None, None, :]

    @pl.when(r == _R - 1)
    def _finish_block():
        z = z_ref[...]
        zq = acc_ref[...]
        zq_ref[...] = zq
        loss_ref[...] += jnp.sum((zq - z) ** 2).reshape(1, 1)

    @pl.when((i == nb - 1) & (r == _R - 1))
    def _finalize():
        n_tok = nb * _BN
        loss_ref[...] = loss_ref[...] / (n_tok * _C)
        usage_ref[...] = usage_ref[...] / n_tok


def kernel(z_map, embed):
    B, Cc, H, W = z_map.shape
    N = B * H * W
    z_tok = jnp.transpose(z_map, (0, 2, 3, 1)).reshape(N, Cc)
    nb = N // _BN
    codes_t, zq_tok, loss, usage = pl.pallas_call(
        _rvq_stage,
        grid=(nb, _R),
        in_specs=[
            pl.BlockSpec((_BN, _C), lambda i, r: (i, 0)),
            pl.BlockSpec((1, _K, _C), lambda i, r: (r, 0, 0)),
        ],
        out_specs=[
            pl.BlockSpec((1, 1, _BN), lambda i, r: (r, 0, i)),
            pl.BlockSpec((_BN, _C), lambda i, r: (i, 0)),
            pl.BlockSpec((1, 1), lambda i, r: (0, 0)),
            pl.BlockSpec((_R, 1, _K), lambda i, r: (0, 0, 0)),
        ],
        out_shape=[
            jax.ShapeDtypeStruct((_R, 1, N), jnp.int32),
            jax.ShapeDtypeStruct((N, Cc), jnp.float32),
            jax.ShapeDtypeStruct((1, 1), jnp.float32),
            jax.ShapeDtypeStruct((_R, 1, _K), jnp.float32),
        ],
        scratch_shapes=[
            pltpu.VMEM((_BN, _C), jnp.float32),
            pltpu.VMEM((_BN, _C), jnp.float32),
        ],
    )(z_tok, embed)
    codes = codes_t.reshape(_R, N).T.reshape(B, H, W, _R)
    zq_map = jnp.transpose(zq_tok.reshape(B, H, W, Cc), (0, 3, 1, 2))
    recon_loss = loss[0, 0]
    usage_out = usage.reshape(_R, _K)
    return codes, zq_map, recon_loss, usage_out


# speedup vs baseline: 1.0492x; 1.0492x over previous
import jax
import jax.numpy as jnp
from jax.experimental import pallas as pl
from jax.experimental.pallas import tpu as pltpu

_R = 6
_K = 512
_C = 192
_BN = 2048


def _rvq_stage(z_ref, cb_ref, codes_ref, zq_ref, loss_ref, usage_ref,
               res_ref):
    i = pl.program_id(0)
    r = pl.program_id(1)
    nb = pl.num_programs(0)

    @pl.when((i == 0) & (r == 0))
    def _init_global():
        loss_ref[...] = jnp.zeros_like(loss_ref)
        usage_ref[...] = jnp.zeros_like(usage_ref)

    @pl.when(r == 0)
    def _init_block():
        res_ref[...] = z_ref[...]

    residual = res_ref[...]
    cb = cb_ref[0]
    cbn = 0.5 * jnp.sum(cb * cb, axis=1, keepdims=True)
    sT = jax.lax.dot_general(
        cb, residual, (((1,), (1,)), ((), ())),
        preferred_element_type=jnp.float32,
        precision=jax.lax.Precision.DEFAULT)
    dT = cbn - sT
    rows = jax.lax.broadcasted_iota(jnp.int32, (_K, _BN), 0)
    dmin = jnp.min(dT, axis=0, keepdims=True)
    idx = jnp.min(jnp.where(dT == dmin, rows, _K), axis=0, keepdims=True)
    onehotT = (rows == idx).astype(jnp.bfloat16)
    cb_hi = cb.astype(jnp.bfloat16)
    rem = cb - cb_hi.astype(jnp.float32)
    cb_mid = rem.astype(jnp.bfloat16)
    cb_lo = (rem - cb_mid.astype(jnp.float32)).astype(jnp.bfloat16)

    def _pick(plane):
        return jax.lax.dot_general(
            onehotT, plane, (((0,), (0,)), ((), ())),
            preferred_element_type=jnp.float32)

    q = _pick(cb_hi) + _pick(cb_mid) + _pick(cb_lo)
    res_ref[...] = residual - q
    codes_ref[0, 0, :] = idx[0]
    ones_col = jnp.ones((_BN, 1), jnp.bfloat16)
    cnt = jax.lax.dot_general(
        onehotT, ones_col, (((1,), (0,)), ((), ())),
        preferred_element_type=jnp.float32)
    usage_ref[pl.ds(r, 1)] += cnt[:, 0][None, None, :]

    @pl.when(r == _R - 1)
    def _finish_block():
        res_final = res_ref[...]
        zq_ref[...] = z_ref[...] - res_final
        loss_ref[...] += jnp.sum(res_final ** 2).reshape(1, 1)

    @pl.when((i == nb - 1) & (r == _R - 1))
    def _finalize():
        n_tok = nb * _BN
        loss_ref[...] = loss_ref[...] / (n_tok * _C)
        usage_ref[...] = usage_ref[...] / n_tok


def kernel(z_map, embed):
    B, Cc, H, W = z_map.shape
    N = B * H * W
    z_tok = jnp.transpose(z_map, (0, 2, 3, 1)).reshape(N, Cc)
    nb = N // _BN
    codes_t, zq_tok, loss, usage = pl.pallas_call(
        _rvq_stage,
        grid=(nb, _R),
        in_specs=[
            pl.BlockSpec((_BN, _C), lambda i, r: (i, 0)),
            pl.BlockSpec((1, _K, _C), lambda i, r: (r, 0, 0)),
        ],
        out_specs=[
            pl.BlockSpec((1, 1, _BN), lambda i, r: (r, 0, i)),
            pl.BlockSpec((_BN, _C), lambda i, r: (i, 0)),
            pl.BlockSpec((1, 1), lambda i, r: (0, 0)),
            pl.BlockSpec((_R, 1, _K), lambda i, r: (0, 0, 0)),
        ],
        out_shape=[
            jax.ShapeDtypeStruct((_R, 1, N), jnp.int32),
            jax.ShapeDtypeStruct((N, Cc), jnp.float32),
            jax.ShapeDtypeStruct((1, 1), jnp.float32),
            jax.ShapeDtypeStruct((_R, 1, _K), jnp.float32),
        ],
        scratch_shapes=[
            pltpu.VMEM((_BN, _C), jnp.float32),
        ],
    )(z_tok, embed)
    codes = codes_t.reshape(_R, N).T.reshape(B, H, W, _R)
    zq_map = jnp.transpose(zq_tok.reshape(B, H, W, Cc), (0, 3, 1, 2))
    recon_loss = loss[0, 0]
    usage_out = usage.reshape(_R, _K)
    return codes, zq_map, recon_loss, usage_out


# speedup vs baseline: 1.1512x; 1.0973x over previous
import jax
import jax.numpy as jnp
from jax.experimental import pallas as pl
from jax.experimental.pallas import tpu as pltpu

_R = 6
_K = 512
_C = 192
_BN = 2048


def _rvq_stage(z_ref, cb_ref, codes_ref, zq_ref, loss_ref, usage_ref,
               res_ref):
    i = pl.program_id(0)
    r = pl.program_id(1)
    nb = pl.num_programs(0)

    @pl.when((i == 0) & (r == 0))
    def _init_global():
        loss_ref[...] = jnp.zeros_like(loss_ref)
        usage_ref[...] = jnp.zeros_like(usage_ref)

    @pl.when(r == 0)
    def _init_block():
        res_ref[...] = z_ref[...]

    residual = res_ref[...]
    cb = cb_ref[0]
    cbn = 0.5 * jnp.sum(cb * cb, axis=1, keepdims=True)
    sT = jax.lax.dot_general(
        cb, residual, (((1,), (1,)), ((), ())),
        preferred_element_type=jnp.float32,
        precision=jax.lax.Precision.DEFAULT)
    dT = cbn - sT
    rows = jax.lax.broadcasted_iota(jnp.int32, (_K, _BN), 0)
    dmin = jnp.min(dT, axis=0, keepdims=True)
    idx = jnp.min(jnp.where(dT == dmin, rows, _K), axis=0, keepdims=True)
    onehotT = (rows == idx).astype(jnp.bfloat16)
    cb_hi = cb.astype(jnp.bfloat16)
    cb_mid = (cb - cb_hi.astype(jnp.float32)).astype(jnp.bfloat16)

    def _pick(plane):
        return jax.lax.dot_general(
            onehotT, plane, (((0,), (0,)), ((), ())),
            preferred_element_type=jnp.float32)

    q = _pick(cb_hi) + _pick(cb_mid)
    res_ref[...] = residual - q
    codes_ref[0, 0, :] = idx[0]
    ones_col = jnp.ones((_BN, 1), jnp.bfloat16)
    cnt = jax.lax.dot_general(
        onehotT, ones_col, (((1,), (0,)), ((), ())),
        preferred_element_type=jnp.float32)
    usage_ref[pl.ds(r, 1)] += cnt[:, 0][None, None, :]

    @pl.when(r == _R - 1)
    def _finish_block():
        res_final = res_ref[...]
        zq_ref[...] = z_ref[...] - res_final
        loss_ref[...] += jnp.sum(res_final ** 2).reshape(1, 1)

    @pl.when((i == nb - 1) & (r == _R - 1))
    def _finalize():
        n_tok = nb * _BN
        loss_ref[...] = loss_ref[...] / (n_tok * _C)
        usage_ref[...] = usage_ref[...] / n_tok


def kernel(z_map, embed):
    B, Cc, H, W = z_map.shape
    N = B * H * W
    z_tok = jnp.transpose(z_map, (0, 2, 3, 1)).reshape(N, Cc)
    nb = N // _BN
    codes_t, zq_tok, loss, usage = pl.pallas_call(
        _rvq_stage,
        grid=(nb, _R),
        in_specs=[
            pl.BlockSpec((_BN, _C), lambda i, r: (i, 0)),
            pl.BlockSpec((1, _K, _C), lambda i, r: (r, 0, 0)),
        ],
        out_specs=[
            pl.BlockSpec((1, 1, _BN), lambda i, r: (r, 0, i)),
            pl.BlockSpec((_BN, _C), lambda i, r: (i, 0)),
            pl.BlockSpec((1, 1), lambda i, r: (0, 0)),
            pl.BlockSpec((_R, 1, _K), lambda i, r: (0, 0, 0)),
        ],
        out_shape=[
            jax.ShapeDtypeStruct((_R, 1, N), jnp.int32),
            jax.ShapeDtypeStruct((N, Cc), jnp.float32),
            jax.ShapeDtypeStruct((1, 1), jnp.float32),
            jax.ShapeDtypeStruct((_R, 1, _K), jnp.float32),
        ],
        scratch_shapes=[
            pltpu.VMEM((_BN, _C), jnp.float32),
        ],
    )(z_tok, embed)
    codes = codes_t.reshape(_R, N).T.reshape(B, H, W, _R)
    zq_map = jnp.transpose(zq_tok.reshape(B, H, W, Cc), (0, 3, 1, 2))
    recon_loss = loss[0, 0]
    usage_out = usage.reshape(_R, _K)
    return codes, zq_map, recon_loss, usage_out


# speedup vs baseline: 1.1832x; 1.0278x over previous
import jax
import jax.numpy as jnp
from jax.experimental import pallas as pl
from jax.experimental.pallas import tpu as pltpu

_R = 6
_K = 512
_C = 192
_BN = 3072


def _rvq_stage(z_ref, cb_ref, codes_ref, zq_ref, loss_ref, usage_ref,
               res_ref):
    i = pl.program_id(0)
    r = pl.program_id(1)
    nb = pl.num_programs(0)

    @pl.when((i == 0) & (r == 0))
    def _init_global():
        loss_ref[...] = jnp.zeros_like(loss_ref)
        usage_ref[...] = jnp.zeros_like(usage_ref)

    @pl.when(r == 0)
    def _init_block():
        res_ref[...] = z_ref[...]

    residual = res_ref[...]
    cb = cb_ref[0]
    cbn = 0.5 * jnp.sum(cb * cb, axis=1, keepdims=True)
    sT = jax.lax.dot_general(
        cb, residual, (((1,), (1,)), ((), ())),
        preferred_element_type=jnp.float32,
        precision=jax.lax.Precision.DEFAULT)
    dT = cbn - sT
    rows = jax.lax.broadcasted_iota(jnp.int32, (_K, _BN), 0)
    dmin = jnp.min(dT, axis=0, keepdims=True)
    idx = jnp.min(jnp.where(dT == dmin, rows, _K), axis=0, keepdims=True)
    onehotT = (rows == idx).astype(jnp.bfloat16)
    cb_hi = cb.astype(jnp.bfloat16)
    cb_mid = (cb - cb_hi.astype(jnp.float32)).astype(jnp.bfloat16)

    def _pick(plane):
        return jax.lax.dot_general(
            onehotT, plane, (((0,), (0,)), ((), ())),
            preferred_element_type=jnp.float32)

    q = _pick(cb_hi) + _pick(cb_mid)
    res_ref[...] = residual - q
    codes_ref[0, 0, :] = idx[0]
    ones_col = jnp.ones((_BN, 1), jnp.bfloat16)
    cnt = jax.lax.dot_general(
        onehotT, ones_col, (((1,), (0,)), ((), ())),
        preferred_element_type=jnp.float32)
    usage_ref[pl.ds(r, 1)] += cnt[:, 0][None, None, :]

    @pl.when(r == _R - 1)
    def _finish_block():
        res_final = res_ref[...]
        zq_ref[...] = z_ref[...] - res_final
        loss_ref[...] += jnp.sum(res_final ** 2).reshape(1, 1)

    @pl.when((i == nb - 1) & (r == _R - 1))
    def _finalize():
        n_tok = nb * _BN
        loss_ref[...] = loss_ref[...] / (n_tok * _C)
        usage_ref[...] = usage_ref[...] / n_tok


def kernel(z_map, embed):
    B, Cc, H, W = z_map.shape
    N = B * H * W
    z_tok = jnp.transpose(z_map, (0, 2, 3, 1)).reshape(N, Cc)
    nb = N // _BN
    codes_t, zq_tok, loss, usage = pl.pallas_call(
        _rvq_stage,
        grid=(nb, _R),
        in_specs=[
            pl.BlockSpec((_BN, _C), lambda i, r: (i, 0)),
            pl.BlockSpec((1, _K, _C), lambda i, r: (r, 0, 0)),
        ],
        out_specs=[
            pl.BlockSpec((1, 1, _BN), lambda i, r: (r, 0, i)),
            pl.BlockSpec((_BN, _C), lambda i, r: (i, 0)),
            pl.BlockSpec((1, 1), lambda i, r: (0, 0)),
            pl.BlockSpec((_R, 1, _K), lambda i, r: (0, 0, 0)),
        ],
        out_shape=[
            jax.ShapeDtypeStruct((_R, 1, N), jnp.int32),
            jax.ShapeDtypeStruct((N, Cc), jnp.float32),
            jax.ShapeDtypeStruct((1, 1), jnp.float32),
            jax.ShapeDtypeStruct((_R, 1, _K), jnp.float32),
        ],
        scratch_shapes=[
            pltpu.VMEM((_BN, _C), jnp.float32),
        ],
    )(z_tok, embed)
    codes = codes_t.reshape(_R, N).T.reshape(B, H, W, _R)
    zq_map = jnp.transpose(zq_tok.reshape(B, H, W, Cc), (0, 3, 1, 2))
    recon_loss = loss[0, 0]
    usage_out = usage.reshape(_R, _K)
    return codes, zq_map, recon_loss, usage_out
